# baseline (speedup 1.0000x reference)
import jax
import jax.numpy as jnp
from jax import lax
from jax.experimental import pallas as pl
from jax.experimental.pallas import tpu as pltpu
from jax.experimental.pallas import tpu_sc as plsc

B = 16384
C = 5
E = 64
EP = 128
NW = 32
CHUNK = 128
NCHUNK = B // (NW * CHUNK)
L = 16


def _body(tgt_hbm, ctx_hbm, ttab_hbm, ctab_hbm, out_hbm,
          tidx, cidx, wrows, crows, outv, sem):
    wid = lax.axis_index("s") * 2 + lax.axis_index("c")
    iota = lax.iota(jnp.int32, L)
    perms = [iota ^ sh for sh in (8, 4, 2, 1)]
    dnums = lax.GatherDimensionNumbers(
        offset_dims=(), collapsed_slice_dims=(0,), start_index_map=(0,))

    def hsum(v):
        for p in perms:
            v = v + lax.gather(v, p[:, None], dnums, slice_sizes=(1,),
                               mode=lax.GatherScatterMode.PROMISE_IN_BOUNDS)
        return v

    for k in range(NCHUNK):
        pltpu.sync_copy(tgt_hbm.at[wid, 0, pl.ds(k * CHUNK, CHUNK)], tidx)
        pltpu.sync_copy(ctx_hbm.at[wid, 0, pl.ds(k * CHUNK * C, CHUNK * C)],
                        cidx)

        cps = [pltpu.async_copy(ttab_hbm.at[tidx], wrows, sem)]
        for j in range(C):
            cps.append(pltpu.async_copy(
                ctab_hbm.at[cidx.at[pl.ds(j * CHUNK, CHUNK)]],
                crows.at[pl.ds(j * CHUNK, CHUNK)], sem))
        for cp in cps:
            cp.wait()

        def row_step(b, carry):
            w = [wrows[b, pl.ds(v * L, L)] for v in range(4)]
            comb = None
            for c in range(C):
                r = b * C + c
                p = w[0] * crows[r, pl.ds(0, L)]
                for v in range(1, 4):
                    p = p + w[v] * crows[r, pl.ds(v * L, L)]
                s = hsum(p) * jnp.float32(0.25)
                comb = s if comb is None else jnp.where(iota == c, s, comb)
            outv[pl.ds(b * C, L)] = comb
            return carry

        lax.fori_loop(0, CHUNK, row_step, 0)
        pltpu.sync_copy(outv.at[pl.ds(0, CHUNK * C)],
                        out_hbm.at[wid, 0, pl.ds(k * CHUNK * C, CHUNK * C)])


@jax.jit
def _run(tgt3, ctx3, ttab_p, ctab_p):
    mesh = plsc.VectorSubcoreMesh(core_axis_name="c", subcore_axis_name="s")
    return pl.kernel(
        _body,
        out_type=jax.ShapeDtypeStruct((NW, 1, B * C // NW), jnp.float32),
        mesh=mesh,
        scratch_types=[
            pltpu.VMEM((CHUNK,), jnp.int32),
            pltpu.VMEM((CHUNK * C,), jnp.int32),
            pltpu.VMEM((CHUNK, EP), jnp.float32),
            pltpu.VMEM((CHUNK * C, EP), jnp.float32),
            pltpu.VMEM((CHUNK * C + L,), jnp.float32),
            pltpu.SemaphoreType.DMA,
        ],
    )(tgt3, ctx3, ttab_p, ctab_p)


def kernel(target, context, target_table, context_table):
    tgt3 = target.reshape(NW, 1, B // NW).astype(jnp.int32)
    ctx3 = context.reshape(NW, 1, B * C // NW).astype(jnp.int32)
    two = jnp.float32(2.0)
    ttab_p = jnp.pad(target_table * two, ((0, 0), (0, EP - E)))
    ctab_p = jnp.pad(context_table * two, ((0, 0), (0, EP - E)))
    out = _run(tgt3, ctx3, ttab_p, ctab_p)
    return out.reshape(B, C)


# speedup vs baseline: 1.4648x; 1.4648x over previous
import jax
import jax.numpy as jnp
from jax import lax
from jax.experimental import pallas as pl
from jax.experimental.pallas import tpu as pltpu
from jax.experimental.pallas import tpu_sc as plsc

B = 16384
C = 5
E = 64
LINE = 128
NW = 32
CHUNK = 128
NCHUNK = B // (NW * CHUNK)
L = 16


def _body(tgt_hbm, ctx_hbm, ttab_hbm, ctab_hbm, out_hbm,
          tidx, cidx, tgidx, cgidx, wrows, crows, outv, sem):
    wid = lax.axis_index("s") * 2 + lax.axis_index("c")
    iota = lax.iota(jnp.int32, L)
    perms = [iota ^ sh for sh in (8, 4, 2, 1)]
    dnums = lax.GatherDimensionNumbers(
        offset_dims=(), collapsed_slice_dims=(0,), start_index_map=(0,))

    def hsum(v):
        for p in perms:
            v = v + lax.gather(v, p[:, None], dnums, slice_sizes=(1,),
                               mode=lax.GatherScatterMode.PROMISE_IN_BOUNDS)
        return v

    for k in range(NCHUNK):
        pltpu.sync_copy(tgt_hbm.at[wid, 0, pl.ds(k * CHUNK, CHUNK)],
                        tidx.at[pl.ds(0, CHUNK)])
        pltpu.sync_copy(ctx_hbm.at[wid, 0, pl.ds(k * CHUNK * C, CHUNK * C)],
                        cidx.at[pl.ds(0, CHUNK * C)])
        for j in range(CHUNK // L):
            tgidx[pl.ds(j * L, L)] = lax.shift_right_logical(
                tidx[pl.ds(j * L, L)], 1)
        for j in range(CHUNK * C // L):
            cgidx[pl.ds(j * L, L)] = lax.shift_right_logical(
                cidx[pl.ds(j * L, L)], 1)

        cps = [pltpu.async_copy(ttab_hbm.at[tgidx], wrows, sem)]
        for j in range(C):
            cps.append(pltpu.async_copy(
                ctab_hbm.at[cgidx.at[pl.ds(j * CHUNK, CHUNK)]],
                crows.at[pl.ds(j * CHUNK, CHUNK)], sem))
        for cp in cps:
            cp.wait()

        def row_step(b, carry):
            th = (tidx[pl.ds(b, L)][0] & 1) * E
            w = [wrows[b, pl.ds(th + v * L, L)] for v in range(4)]
            comb = None
            for c in range(C):
                r = b * C + c
                ch = (cidx[pl.ds(r, L)][0] & 1) * E
                p = w[0] * crows[r, pl.ds(ch, L)]
                for v in range(1, 4):
                    p = p + w[v] * crows[r, pl.ds(ch + v * L, L)]
                s = hsum(p)
                comb = s if comb is None else jnp.where(iota == c, s, comb)
            outv[pl.ds(b * C, L)] = comb
            return carry

        lax.fori_loop(0, CHUNK, row_step, 0)
        pltpu.sync_copy(outv.at[pl.ds(0, CHUNK * C)],
                        out_hbm.at[wid, 0, pl.ds(k * CHUNK * C, CHUNK * C)])


@jax.jit
def _run(tgt3, ctx3, ttab_l, ctab_l):
    mesh = plsc.VectorSubcoreMesh(core_axis_name="c", subcore_axis_name="s")
    return pl.kernel(
        _body,
        out_type=jax.ShapeDtypeStruct((NW, 1, B * C // NW), jnp.float32),
        mesh=mesh,
        scratch_types=[
            pltpu.VMEM((CHUNK + L,), jnp.int32),
            pltpu.VMEM((CHUNK * C + L,), jnp.int32),
            pltpu.VMEM((CHUNK,), jnp.int32),
            pltpu.VMEM((CHUNK * C,), jnp.int32),
            pltpu.VMEM((CHUNK, LINE), jnp.float32),
            pltpu.VMEM((CHUNK * C, LINE), jnp.float32),
            pltpu.VMEM((CHUNK * C + L,), jnp.float32),
            pltpu.SemaphoreType.DMA,
        ],
    )(tgt3, ctx3, ttab_l, ctab_l)


def kernel(target, context, target_table, context_table):
    tgt3 = target.reshape(NW, 1, B // NW).astype(jnp.int32)
    ctx3 = context.reshape(NW, 1, B * C // NW).astype(jnp.int32)
    out = _run(tgt3, ctx3,
               target_table.reshape(1000000 // 2, LINE),
               context_table.reshape(1000000 // 2, LINE))
    return out.reshape(B, C)


# speedup vs baseline: 1.5712x; 1.0727x over previous
import jax
import jax.numpy as jnp
from jax import lax
from jax.experimental import pallas as pl
from jax.experimental.pallas import tpu as pltpu
from jax.experimental.pallas import tpu_sc as plsc

B = 16384
C = 5
E = 64
EP = 128
NW = 32
CHUNK = 128
NCHUNK = B // (NW * CHUNK)
L = 16


def _body(tgt_hbm, ctx_hbm, ttab_hbm, ctab_hbm, out_hbm,
          tidx, cidx, wrows, crows, outv, sem):
    wid = lax.axis_index("s") * 2 + lax.axis_index("c")
    iota = lax.iota(jnp.int32, L)
    perms = [iota ^ sh for sh in (8, 4, 2, 1)]
    dnums = lax.GatherDimensionNumbers(
        offset_dims=(), collapsed_slice_dims=(0,), start_index_map=(0,))

    def hsum(v):
        for p in perms:
            v = v + lax.gather(v, p[:, None], dnums, slice_sizes=(1,),
                               mode=lax.GatherScatterMode.PROMISE_IN_BOUNDS)
        return v

    for k in range(NCHUNK):
        pltpu.sync_copy(tgt_hbm.at[wid, 0, pl.ds(k * CHUNK, CHUNK)], tidx)
        pltpu.sync_copy(ctx_hbm.at[wid, 0, pl.ds(k * CHUNK * C, CHUNK * C)],
                        cidx)

        cps = [pltpu.async_copy(ttab_hbm.at[tidx], wrows, sem)]
        for j in range(C):
            cps.append(pltpu.async_copy(
                ctab_hbm.at[cidx.at[pl.ds(j * CHUNK, CHUNK)]],
                crows.at[pl.ds(j * CHUNK, CHUNK)], sem))
        for cp in cps:
            cp.wait()

        def row_step(b, carry):
            w = [wrows[b, pl.ds(v * L, L)] for v in range(4)]
            comb = None
            for c in range(C):
                r = b * C + c
                p = w[0] * crows[r, pl.ds(0, L)]
                for v in range(1, 4):
                    p = p + w[v] * crows[r, pl.ds(v * L, L)]
                s = hsum(p)
                comb = s if comb is None else jnp.where(iota == c, s, comb)
            outv[pl.ds(b * C, L)] = comb
            return carry

        lax.fori_loop(0, CHUNK, row_step, 0)
        pltpu.sync_copy(outv.at[pl.ds(0, CHUNK * C)],
                        out_hbm.at[wid, 0, pl.ds(k * CHUNK * C, CHUNK * C)])


@jax.jit
def _run(tgt3, ctx3, ttab_p, ctab_p):
    mesh = plsc.VectorSubcoreMesh(core_axis_name="c", subcore_axis_name="s")
    return pl.kernel(
        _body,
        out_type=jax.ShapeDtypeStruct((NW, 1, B * C // NW), jnp.float32),
        mesh=mesh,
        scratch_types=[
            pltpu.VMEM((CHUNK,), jnp.int32),
            pltpu.VMEM((CHUNK * C,), jnp.int32),
            pltpu.VMEM((CHUNK, EP), jnp.float32),
            pltpu.VMEM((CHUNK * C, EP), jnp.float32),
            pltpu.VMEM((CHUNK * C + L,), jnp.float32),
            pltpu.SemaphoreType.DMA,
        ],
    )(tgt3, ctx3, ttab_p, ctab_p)


def kernel(target, context, target_table, context_table):
    tgt3 = target.reshape(NW, 1, B // NW).astype(jnp.int32)
    ctx3 = context.reshape(NW, 1, B * C // NW).astype(jnp.int32)
    ttab_p = jnp.pad(target_table, ((0, 0), (0, EP - E)))
    ctab_p = jnp.pad(context_table, ((0, 0), (0, EP - E)))
    out = _run(tgt3, ctx3, ttab_p, ctab_p)
    return out.reshape(B, C)
